# baseline (speedup 1.0000x reference)
import functools

import jax
import jax.numpy as jnp
from jax import lax
from jax.experimental import pallas as pl
from jax.experimental.pallas import tpu as pltpu
from jax.experimental.pallas import tpu_sc as plsc

N = 10000
E = 320000
N_PAD = 10112
NW = 32
K = 128
CH = 79
E_PAD = NW * CH * K
ROWS_PER_TILE = N_PAD // 16
N_PAD_DEG = 10240
RPT_DEG = N_PAD_DEG // 16

K2 = 56
CH2 = 180
E_PAD2 = NW * CH2 * K2
G_CHUNKS = NW * CH2
CHA = 120
CHB = 240
NSLOT = 6


def _norm(deg):
    return jnp.where(deg > 0, lax.rsqrt(jnp.maximum(deg, 1.0)), 0.0)


def _sc_mesh():
    return plsc.VectorSubcoreMesh(core_axis_name="c", subcore_axis_name="s")


def _sc_degrees(srcw, dstw, ones_col, zeros_col):
    @functools.partial(
        pl.kernel,
        out_type=(
            jax.ShapeDtypeStruct((2, N_PAD_DEG), jnp.float32),
            jax.ShapeDtypeStruct((2, N_PAD_DEG), jnp.float32),
        ),
        mesh=_sc_mesh(),
        scratch_types=[
            pltpu.VMEM((CH, K), jnp.int32),
            pltpu.VMEM((CH, K), jnp.int32),
            pltpu.VMEM((K,), jnp.float32),
            pltpu.VMEM_SHARED((N_PAD_DEG,), jnp.float32),
            pltpu.VMEM_SHARED((N_PAD_DEG,), jnp.float32),
        ],
    )
    def k(srcw_hbm, dstw_hbm, ones_hbm, zeros_hbm, dego_hbm, degi_hbm,
          sidx, didx, ones_v, acc_o, acc_i):
        c = lax.axis_index("c")
        s = lax.axis_index("s")
        w = s * 2 + c
        pltpu.sync_copy(zeros_hbm, acc_o.at[pl.ds(s * RPT_DEG, RPT_DEG)])
        pltpu.sync_copy(zeros_hbm, acc_i.at[pl.ds(s * RPT_DEG, RPT_DEG)])
        pltpu.sync_copy(ones_hbm, ones_v)
        pltpu.sync_copy(srcw_hbm.at[w], sidx)
        pltpu.sync_copy(dstw_hbm.at[w], didx)
        plsc.subcore_barrier()

        def body(j, carry):
            pltpu.sync_copy(ones_v, acc_o.at[sidx.at[j]], add=True)
            pltpu.sync_copy(ones_v, acc_i.at[didx.at[j]], add=True)
            return carry

        lax.fori_loop(0, CH, body, 0)
        plsc.subcore_barrier()
        r0 = s * RPT_DEG
        pltpu.sync_copy(acc_o.at[pl.ds(r0, RPT_DEG)],
                        dego_hbm.at[c, pl.ds(r0, RPT_DEG)])
        pltpu.sync_copy(acc_i.at[pl.ds(r0, RPT_DEG)],
                        degi_hbm.at[c, pl.ds(r0, RPT_DEG)])

    return k(srcw, dstw, ones_col, zeros_col)


def _sc_aggregate(P, edges, zeros_f, F):
    @functools.partial(
        pl.kernel,
        out_type=jax.ShapeDtypeStruct((2, N_PAD, F), jnp.float32),
        mesh=_sc_mesh(),
        scratch_types=[
            pltpu.VMEM((NSLOT, 1, K2), jnp.int32),
            pltpu.VMEM((NSLOT, 1, K2), jnp.int32),
            pltpu.VMEM((NSLOT, K2, F), jnp.float32),
            pltpu.VMEM_SHARED((N_PAD, F), jnp.float32),
        ] + [pltpu.SemaphoreType.DMA] * (3 * NSLOT),
    )
    def k(p_hbm, srcw_hbm, dstw_hbm, zeros_hbm, out_hbm,
          sidx, didx, rows, acc, *sems):
        sem_i = sems[0:NSLOT]
        sem_g = sems[NSLOT:2 * NSLOT]
        sem_s = sems[2 * NSLOT:3 * NSLOT]
        c = lax.axis_index("c")
        s = lax.axis_index("s")
        g0 = jnp.where(c == 0, s * CHA, 16 * CHA + s * CHB)
        ch_loc = jnp.where(c == 0, CHA, CHB)

        def fetch_idx(j, slot):
            pltpu.async_copy(srcw_hbm.at[g0 + j], sidx.at[slot], sem_i[slot])
            pltpu.async_copy(dstw_hbm.at[g0 + j], didx.at[slot], sem_i[slot])

        def wait_idx(j, slot):
            pltpu.make_async_copy(srcw_hbm.at[g0 + j], sidx.at[slot],
                                  sem_i[slot]).wait()
            pltpu.make_async_copy(dstw_hbm.at[g0 + j], didx.at[slot],
                                  sem_i[slot]).wait()

        def start_gather(slot):
            pltpu.async_copy(p_hbm.at[sidx.at[slot, 0]], rows.at[slot],
                             sem_g[slot])

        def wait_gather(slot):
            pltpu.make_async_copy(p_hbm.at[sidx.at[slot, 0]], rows.at[slot],
                                  sem_g[slot]).wait()

        def start_scatter(slot):
            pltpu.async_copy(rows.at[slot], acc.at[didx.at[slot, 0]],
                             sem_s[slot], add=True)

        def wait_scatter(slot):
            pltpu.make_async_copy(rows.at[slot], acc.at[didx.at[slot, 0]],
                                  sem_s[slot]).wait()

        pltpu.sync_copy(zeros_hbm,
                        acc.at[pl.ds(s * ROWS_PER_TILE, ROWS_PER_TILE)])
        for t in range(4):
            fetch_idx(t, t)
        for t in range(3):
            wait_idx(t, t)
            start_gather(t)
        plsc.subcore_barrier()

        def group(g, carry):
            for b in range(NSLOT):
                j = g * NSLOT + b
                wait_gather(b)
                start_scatter(b)

                @pl.when(j >= 2)
                def _():
                    wait_scatter((b + 4) % NSLOT)

                @pl.when(j + 4 < ch_loc)
                def _():
                    fetch_idx(j + 4, (b + 4) % NSLOT)

                @pl.when(j + 3 < ch_loc)
                def _():
                    wait_idx(j + 3, (b + 3) % NSLOT)
                    start_gather((b + 3) % NSLOT)

            return carry

        lax.fori_loop(0, ch_loc // NSLOT, group, 0)
        wait_scatter((CHA - 2) % NSLOT)
        wait_scatter((CHA - 1) % NSLOT)
        plsc.subcore_barrier()
        r0 = s * ROWS_PER_TILE
        pltpu.sync_copy(acc.at[pl.ds(r0, ROWS_PER_TILE)],
                        out_hbm.at[c, pl.ds(r0, ROWS_PER_TILE)])

    return k(P, edges[0], edges[1], zeros_f)


BM = 256
GRID = (N + BM - 1) // BM


def _tc_stage1(features, dego, W1):
    def body(f_ref, d_ref, w_ref, o_ref):
        deg = d_ref[0] + d_ref[1]
        ns = _norm(deg)
        o_ref[...] = jnp.dot(f_ref[...] * ns, w_ref[...],
                             preferred_element_type=jnp.float32)

    return pl.pallas_call(
        body,
        grid=(GRID,),
        in_specs=[
            pl.BlockSpec((BM, 128), lambda i: (i, 0)),
            pl.BlockSpec((2, BM, 1), lambda i: (0, i, 0)),
            pl.BlockSpec((128, 128), lambda i: (0, 0)),
        ],
        out_specs=pl.BlockSpec((BM, 128), lambda i: (i, 0)),
        out_shape=jax.ShapeDtypeStruct((N, 128), jnp.float32),
    )(features, dego, W1)


def _tc_stage2(a1p, dego, degi, W2, b1r):
    def body(a_ref, do_ref, di_ref, w_ref, b_ref, o_ref):
        a = a_ref[0] + a_ref[1]
        nd = _norm(di_ref[0] + di_ref[1])
        ns = _norm(do_ref[0] + do_ref[1])
        h = jnp.maximum(a * nd + b_ref[...], 0.0)
        o_ref[...] = jnp.dot(h * ns, w_ref[...],
                             preferred_element_type=jnp.float32)

    return pl.pallas_call(
        body,
        grid=(GRID,),
        in_specs=[
            pl.BlockSpec((2, BM, 128), lambda i: (0, i, 0)),
            pl.BlockSpec((2, BM, 1), lambda i: (0, i, 0)),
            pl.BlockSpec((2, BM, 1), lambda i: (0, i, 0)),
            pl.BlockSpec((128, 128), lambda i: (0, 0)),
            pl.BlockSpec((1, 128), lambda i: (0, 0)),
        ],
        out_specs=pl.BlockSpec((BM, 128), lambda i: (i, 0)),
        out_shape=jax.ShapeDtypeStruct((N, 128), jnp.float32),
    )(a1p, dego, degi, W2, b1r)


def _tc_stage3(a2p, degi, b2r):
    def body(a_ref, di_ref, b_ref, o_ref):
        a = a_ref[0, :, :64] + a_ref[1, :, :64]
        nd = _norm(di_ref[0] + di_ref[1])
        o_ref[...] = a * nd + b_ref[...]

    return pl.pallas_call(
        body,
        grid=(GRID,),
        in_specs=[
            pl.BlockSpec((2, BM, 128), lambda i: (0, i, 0)),
            pl.BlockSpec((2, BM, 1), lambda i: (0, i, 0)),
            pl.BlockSpec((1, 64), lambda i: (0, 0)),
        ],
        out_specs=pl.BlockSpec((BM, 64), lambda i: (i, 0)),
        out_shape=jax.ShapeDtypeStruct((N, 64), jnp.float32),
    )(a2p, degi, b2r)


def kernel(features, edge_index, W1, b1, W2, b2):
    src = edge_index[0]
    dst = edge_index[1]
    n_extra = E_PAD - E
    pad_scr = N + (jnp.arange(n_extra, dtype=jnp.int32) % (N_PAD_DEG - N))
    src_d = jnp.concatenate([src, pad_scr]).reshape(NW, CH, K)
    dst_p = jnp.concatenate([dst, pad_scr]).reshape(NW, CH, K)

    n2 = E_PAD2 - E
    pad_scr2 = N + (jnp.arange(n2, dtype=jnp.int32) % (N_PAD - N))
    src2 = jnp.concatenate(
        [src, jnp.zeros((n2,), jnp.int32)]).reshape(G_CHUNKS, 1, K2)
    dst2 = jnp.concatenate([dst, pad_scr2]).reshape(G_CHUNKS, 1, K2)
    edges2 = (src2, dst2)

    ones_col = jnp.ones((K,), jnp.float32)
    zeros_col = jnp.zeros((RPT_DEG,), jnp.float32)
    zeros128 = jnp.zeros((ROWS_PER_TILE, 128), jnp.float32)
    zeros64 = jnp.zeros((ROWS_PER_TILE, 64), jnp.float32)

    dego, degi = _sc_degrees(src_d, dst_p, ones_col, zeros_col)
    dego = dego[:, :, None]
    degi = degi[:, :, None]

    w2p = jnp.concatenate([W2, jnp.zeros((128, 64), jnp.float32)], axis=1)

    p1 = _tc_stage1(features, dego, W1)
    a1p = _sc_aggregate(p1, edges2, zeros128, 128)
    p2 = _tc_stage2(a1p, dego, degi, w2p, jnp.reshape(b1, (1, 128)))
    a2p = _sc_aggregate(p2, edges2, zeros128, 128)
    out = _tc_stage3(a2p, degi, jnp.reshape(b2, (1, 64)))
    return out


# speedup vs baseline: 1.1369x; 1.1369x over previous
import functools

import jax
import jax.numpy as jnp
from jax import lax
from jax.experimental import pallas as pl
from jax.experimental.pallas import tpu as pltpu
from jax.experimental.pallas import tpu_sc as plsc

N = 10000
E = 320000
N_PAD = 10112
NW = 32
K = 128
CH = 79
E_PAD = NW * CH * K
ROWS_PER_TILE = N_PAD // 16
N_PAD_DEG = 10240
RPT_DEG = N_PAD_DEG // 16

K2 = 56
CH2 = 180
E_PAD2 = NW * CH2 * K2
G_CHUNKS = NW * CH2
CHA = 240
CHB = 120
NSLOT = 6


def _norm(deg):
    return jnp.where(deg > 0, lax.rsqrt(jnp.maximum(deg, 1.0)), 0.0)


def _sc_mesh():
    return plsc.VectorSubcoreMesh(core_axis_name="c", subcore_axis_name="s")


def _sc_degrees(srcw, dstw, ones_col, zeros_col):
    @functools.partial(
        pl.kernel,
        out_type=(
            jax.ShapeDtypeStruct((2, N_PAD_DEG), jnp.float32),
            jax.ShapeDtypeStruct((2, N_PAD_DEG), jnp.float32),
        ),
        mesh=_sc_mesh(),
        scratch_types=[
            pltpu.VMEM((CH, K), jnp.int32),
            pltpu.VMEM((CH, K), jnp.int32),
            pltpu.VMEM((K,), jnp.float32),
            pltpu.VMEM_SHARED((N_PAD_DEG,), jnp.float32),
            pltpu.VMEM_SHARED((N_PAD_DEG,), jnp.float32),
        ],
    )
    def k(srcw_hbm, dstw_hbm, ones_hbm, zeros_hbm, dego_hbm, degi_hbm,
          sidx, didx, ones_v, acc_o, acc_i):
        c = lax.axis_index("c")
        s = lax.axis_index("s")
        w = s * 2 + c
        pltpu.sync_copy(zeros_hbm, acc_o.at[pl.ds(s * RPT_DEG, RPT_DEG)])
        pltpu.sync_copy(zeros_hbm, acc_i.at[pl.ds(s * RPT_DEG, RPT_DEG)])
        pltpu.sync_copy(ones_hbm, ones_v)
        pltpu.sync_copy(srcw_hbm.at[w], sidx)
        pltpu.sync_copy(dstw_hbm.at[w], didx)
        plsc.subcore_barrier()

        def body(j, carry):
            pltpu.sync_copy(ones_v, acc_o.at[sidx.at[j]], add=True)
            pltpu.sync_copy(ones_v, acc_i.at[didx.at[j]], add=True)
            return carry

        lax.fori_loop(0, CH, body, 0)
        plsc.subcore_barrier()
        r0 = s * RPT_DEG
        pltpu.sync_copy(acc_o.at[pl.ds(r0, RPT_DEG)],
                        dego_hbm.at[c, pl.ds(r0, RPT_DEG)])
        pltpu.sync_copy(acc_i.at[pl.ds(r0, RPT_DEG)],
                        degi_hbm.at[c, pl.ds(r0, RPT_DEG)])

    return k(srcw, dstw, ones_col, zeros_col)


def _sc_aggregate(P, edges, zeros_f, F):
    @functools.partial(
        pl.kernel,
        out_type=jax.ShapeDtypeStruct((2, N_PAD, F), jnp.float32),
        mesh=_sc_mesh(),
        scratch_types=[
            pltpu.VMEM((NSLOT, 1, K2), jnp.int32),
            pltpu.VMEM((NSLOT, 1, K2), jnp.int32),
            pltpu.VMEM((NSLOT, K2, F), jnp.float32),
            pltpu.VMEM_SHARED((N_PAD, F), jnp.float32),
        ] + [pltpu.SemaphoreType.DMA] * (3 * NSLOT),
    )
    def k(p_hbm, srcw_hbm, dstw_hbm, zeros_hbm, out_hbm,
          sidx, didx, rows, acc, *sems):
        sem_i = sems[0:NSLOT]
        sem_g = sems[NSLOT:2 * NSLOT]
        sem_s = sems[2 * NSLOT:3 * NSLOT]
        c = lax.axis_index("c")
        s = lax.axis_index("s")
        g0 = jnp.where(c == 0, s * CHA, 16 * CHA + s * CHB)
        ch_loc = jnp.where(c == 0, CHA, CHB)

        def fetch_idx(j, slot):
            pltpu.async_copy(srcw_hbm.at[g0 + j], sidx.at[slot], sem_i[slot])
            pltpu.async_copy(dstw_hbm.at[g0 + j], didx.at[slot], sem_i[slot])

        def wait_idx(j, slot):
            pltpu.make_async_copy(srcw_hbm.at[g0 + j], sidx.at[slot],
                                  sem_i[slot]).wait()
            pltpu.make_async_copy(dstw_hbm.at[g0 + j], didx.at[slot],
                                  sem_i[slot]).wait()

        def start_gather(slot):
            pltpu.async_copy(p_hbm.at[sidx.at[slot, 0]], rows.at[slot],
                             sem_g[slot])

        def wait_gather(slot):
            pltpu.make_async_copy(p_hbm.at[sidx.at[slot, 0]], rows.at[slot],
                                  sem_g[slot]).wait()

        def start_scatter(slot):
            pltpu.async_copy(rows.at[slot], acc.at[didx.at[slot, 0]],
                             sem_s[slot], add=True)

        def wait_scatter(slot):
            pltpu.make_async_copy(rows.at[slot], acc.at[didx.at[slot, 0]],
                                  sem_s[slot]).wait()

        pltpu.sync_copy(zeros_hbm,
                        acc.at[pl.ds(s * ROWS_PER_TILE, ROWS_PER_TILE)])
        for t in range(4):
            fetch_idx(t, t)
        for t in range(3):
            wait_idx(t, t)
            start_gather(t)
        plsc.subcore_barrier()

        def group(g, carry):
            for b in range(NSLOT):
                j = g * NSLOT + b
                wait_gather(b)
                start_scatter(b)

                @pl.when(j >= 2)
                def _():
                    wait_scatter((b + 4) % NSLOT)

                @pl.when(j + 4 < ch_loc)
                def _():
                    fetch_idx(j + 4, (b + 4) % NSLOT)

                @pl.when(j + 3 < ch_loc)
                def _():
                    wait_idx(j + 3, (b + 3) % NSLOT)
                    start_gather((b + 3) % NSLOT)

            return carry

        lax.fori_loop(0, ch_loc // NSLOT, group, 0)
        wait_scatter((CHA - 2) % NSLOT)
        wait_scatter((CHA - 1) % NSLOT)
        plsc.subcore_barrier()
        r0 = s * ROWS_PER_TILE
        pltpu.sync_copy(acc.at[pl.ds(r0, ROWS_PER_TILE)],
                        out_hbm.at[c, pl.ds(r0, ROWS_PER_TILE)])

    return k(P, edges[0], edges[1], zeros_f)


BM = 256
GRID = (N + BM - 1) // BM


def _tc_stage1(features, dego, W1):
    def body(f_ref, d_ref, w_ref, o_ref):
        deg = d_ref[0] + d_ref[1]
        ns = _norm(deg)
        o_ref[...] = jnp.dot(f_ref[...] * ns, w_ref[...],
                             preferred_element_type=jnp.float32)

    return pl.pallas_call(
        body,
        grid=(GRID,),
        in_specs=[
            pl.BlockSpec((BM, 128), lambda i: (i, 0)),
            pl.BlockSpec((2, BM, 1), lambda i: (0, i, 0)),
            pl.BlockSpec((128, 128), lambda i: (0, 0)),
        ],
        out_specs=pl.BlockSpec((BM, 128), lambda i: (i, 0)),
        out_shape=jax.ShapeDtypeStruct((N, 128), jnp.float32),
    )(features, dego, W1)


def _tc_stage2(a1p, dego, degi, W2, b1r):
    def body(a_ref, do_ref, di_ref, w_ref, b_ref, o_ref):
        a = a_ref[0] + a_ref[1]
        nd = _norm(di_ref[0] + di_ref[1])
        ns = _norm(do_ref[0] + do_ref[1])
        h = jnp.maximum(a * nd + b_ref[...], 0.0)
        o_ref[...] = jnp.dot(h * ns, w_ref[...],
                             preferred_element_type=jnp.float32)

    return pl.pallas_call(
        body,
        grid=(GRID,),
        in_specs=[
            pl.BlockSpec((2, BM, 128), lambda i: (0, i, 0)),
            pl.BlockSpec((2, BM, 1), lambda i: (0, i, 0)),
            pl.BlockSpec((2, BM, 1), lambda i: (0, i, 0)),
            pl.BlockSpec((128, 128), lambda i: (0, 0)),
            pl.BlockSpec((1, 128), lambda i: (0, 0)),
        ],
        out_specs=pl.BlockSpec((BM, 128), lambda i: (i, 0)),
        out_shape=jax.ShapeDtypeStruct((N, 128), jnp.float32),
    )(a1p, dego, degi, W2, b1r)


def _tc_stage3(a2p, degi, b2r):
    def body(a_ref, di_ref, b_ref, o_ref):
        a = a_ref[0, :, :64] + a_ref[1, :, :64]
        nd = _norm(di_ref[0] + di_ref[1])
        o_ref[...] = a * nd + b_ref[...]

    return pl.pallas_call(
        body,
        grid=(GRID,),
        in_specs=[
            pl.BlockSpec((2, BM, 128), lambda i: (0, i, 0)),
            pl.BlockSpec((2, BM, 1), lambda i: (0, i, 0)),
            pl.BlockSpec((1, 64), lambda i: (0, 0)),
        ],
        out_specs=pl.BlockSpec((BM, 64), lambda i: (i, 0)),
        out_shape=jax.ShapeDtypeStruct((N, 64), jnp.float32),
    )(a2p, degi, b2r)


def kernel(features, edge_index, W1, b1, W2, b2):
    src = edge_index[0]
    dst = edge_index[1]
    n_extra = E_PAD - E
    pad_scr = N + (jnp.arange(n_extra, dtype=jnp.int32) % (N_PAD_DEG - N))
    src_d = jnp.concatenate([src, pad_scr]).reshape(NW, CH, K)
    dst_p = jnp.concatenate([dst, pad_scr]).reshape(NW, CH, K)

    n2 = E_PAD2 - E
    pad_scr2 = N + (jnp.arange(n2, dtype=jnp.int32) % (N_PAD - N))
    src2 = jnp.concatenate(
        [src, jnp.zeros((n2,), jnp.int32)]).reshape(G_CHUNKS, 1, K2)
    dst2 = jnp.concatenate([dst, pad_scr2]).reshape(G_CHUNKS, 1, K2)
    edges2 = (src2, dst2)

    ones_col = jnp.ones((K,), jnp.float32)
    zeros_col = jnp.zeros((RPT_DEG,), jnp.float32)
    zeros128 = jnp.zeros((ROWS_PER_TILE, 128), jnp.float32)
    zeros64 = jnp.zeros((ROWS_PER_TILE, 64), jnp.float32)

    dego, degi = _sc_degrees(src_d, dst_p, ones_col, zeros_col)
    dego = dego[:, :, None]
    degi = degi[:, :, None]

    w2p = jnp.concatenate([W2, jnp.zeros((128, 64), jnp.float32)], axis=1)

    p1 = _tc_stage1(features, dego, W1)
    a1p = _sc_aggregate(p1, edges2, zeros128, 128)
    p2 = _tc_stage2(a1p, dego, degi, w2p, jnp.reshape(b1, (1, 128)))
    a2p = _sc_aggregate(p2, edges2, zeros128, 128)
    out = _tc_stage3(a2p, degi, jnp.reshape(b2, (1, 64)))
    return out


# speedup vs baseline: 1.1956x; 1.0516x over previous
import functools

import jax
import jax.numpy as jnp
from jax import lax
from jax.experimental import pallas as pl
from jax.experimental.pallas import tpu as pltpu
from jax.experimental.pallas import tpu_sc as plsc

N = 10000
E = 320000
N_PAD = 10112
NW = 32
K = 128
CH = 79
E_PAD = NW * CH * K
ROWS_PER_TILE = N_PAD // 16
N_PAD_DEG = 10240
RPT_DEG = N_PAD_DEG // 16

K2 = 56
CH2 = 180
E_PAD2 = NW * CH2 * K2
G_CHUNKS = NW * CH2
CHA = 210
CHB = 150
NSLOT = 6


def _norm(deg):
    return jnp.where(deg > 0, lax.rsqrt(jnp.maximum(deg, 1.0)), 0.0)


def _sc_mesh():
    return plsc.VectorSubcoreMesh(core_axis_name="c", subcore_axis_name="s")


def _sc_degrees(srcw, dstw, ones_col, zeros_col):
    @functools.partial(
        pl.kernel,
        out_type=(
            jax.ShapeDtypeStruct((2, N_PAD_DEG), jnp.float32),
            jax.ShapeDtypeStruct((2, N_PAD_DEG), jnp.float32),
        ),
        mesh=_sc_mesh(),
        scratch_types=[
            pltpu.VMEM((CH, K), jnp.int32),
            pltpu.VMEM((CH, K), jnp.int32),
            pltpu.VMEM((K,), jnp.float32),
            pltpu.VMEM_SHARED((N_PAD_DEG,), jnp.float32),
            pltpu.VMEM_SHARED((N_PAD_DEG,), jnp.float32),
        ],
    )
    def k(srcw_hbm, dstw_hbm, ones_hbm, zeros_hbm, dego_hbm, degi_hbm,
          sidx, didx, ones_v, acc_o, acc_i):
        c = lax.axis_index("c")
        s = lax.axis_index("s")
        w = s * 2 + c
        pltpu.sync_copy(zeros_hbm, acc_o.at[pl.ds(s * RPT_DEG, RPT_DEG)])
        pltpu.sync_copy(zeros_hbm, acc_i.at[pl.ds(s * RPT_DEG, RPT_DEG)])
        pltpu.sync_copy(ones_hbm, ones_v)
        pltpu.sync_copy(srcw_hbm.at[w], sidx)
        pltpu.sync_copy(dstw_hbm.at[w], didx)
        plsc.subcore_barrier()

        def body(j, carry):
            pltpu.sync_copy(ones_v, acc_o.at[sidx.at[j]], add=True)
            pltpu.sync_copy(ones_v, acc_i.at[didx.at[j]], add=True)
            return carry

        lax.fori_loop(0, CH, body, 0)
        plsc.subcore_barrier()
        r0 = s * RPT_DEG
        pltpu.sync_copy(acc_o.at[pl.ds(r0, RPT_DEG)],
                        dego_hbm.at[c, pl.ds(r0, RPT_DEG)])
        pltpu.sync_copy(acc_i.at[pl.ds(r0, RPT_DEG)],
                        degi_hbm.at[c, pl.ds(r0, RPT_DEG)])

    return k(srcw, dstw, ones_col, zeros_col)


def _sc_aggregate(P, edges, zeros_f, F):
    @functools.partial(
        pl.kernel,
        out_type=jax.ShapeDtypeStruct((2, N_PAD, F), jnp.float32),
        mesh=_sc_mesh(),
        scratch_types=[
            pltpu.VMEM((NSLOT, 1, K2), jnp.int32),
            pltpu.VMEM((NSLOT, 1, K2), jnp.int32),
            pltpu.VMEM((NSLOT, K2, F), jnp.float32),
            pltpu.VMEM_SHARED((N_PAD, F), jnp.float32),
        ] + [pltpu.SemaphoreType.DMA] * (3 * NSLOT),
    )
    def k(p_hbm, srcw_hbm, dstw_hbm, zeros_hbm, out_hbm,
          sidx, didx, rows, acc, *sems):
        sem_i = sems[0:NSLOT]
        sem_g = sems[NSLOT:2 * NSLOT]
        sem_s = sems[2 * NSLOT:3 * NSLOT]
        c = lax.axis_index("c")
        s = lax.axis_index("s")
        g0 = jnp.where(c == 0, s * CHA, 16 * CHA + s * CHB)
        ch_loc = jnp.where(c == 0, CHA, CHB)

        def fetch_idx(j, slot):
            pltpu.async_copy(srcw_hbm.at[g0 + j], sidx.at[slot], sem_i[slot])
            pltpu.async_copy(dstw_hbm.at[g0 + j], didx.at[slot], sem_i[slot])

        def wait_idx(j, slot):
            pltpu.make_async_copy(srcw_hbm.at[g0 + j], sidx.at[slot],
                                  sem_i[slot]).wait()
            pltpu.make_async_copy(dstw_hbm.at[g0 + j], didx.at[slot],
                                  sem_i[slot]).wait()

        def start_gather(slot):
            pltpu.async_copy(p_hbm.at[sidx.at[slot, 0]], rows.at[slot],
                             sem_g[slot])

        def wait_gather(slot):
            pltpu.make_async_copy(p_hbm.at[sidx.at[slot, 0]], rows.at[slot],
                                  sem_g[slot]).wait()

        def start_scatter(slot):
            pltpu.async_copy(rows.at[slot], acc.at[didx.at[slot, 0]],
                             sem_s[slot], add=True)

        def wait_scatter(slot):
            pltpu.make_async_copy(rows.at[slot], acc.at[didx.at[slot, 0]],
                                  sem_s[slot]).wait()

        pltpu.sync_copy(zeros_hbm,
                        acc.at[pl.ds(s * ROWS_PER_TILE, ROWS_PER_TILE)])
        for t in range(4):
            fetch_idx(t, t)
        for t in range(3):
            wait_idx(t, t)
            start_gather(t)
        plsc.subcore_barrier()

        def group(g, carry):
            for b in range(NSLOT):
                j = g * NSLOT + b
                wait_gather(b)
                start_scatter(b)

                @pl.when(j >= 2)
                def _():
                    wait_scatter((b + 4) % NSLOT)

                @pl.when(j + 4 < ch_loc)
                def _():
                    fetch_idx(j + 4, (b + 4) % NSLOT)

                @pl.when(j + 3 < ch_loc)
                def _():
                    wait_idx(j + 3, (b + 3) % NSLOT)
                    start_gather((b + 3) % NSLOT)

            return carry

        lax.fori_loop(0, ch_loc // NSLOT, group, 0)
        wait_scatter((CHA - 2) % NSLOT)
        wait_scatter((CHA - 1) % NSLOT)
        plsc.subcore_barrier()
        r0 = s * ROWS_PER_TILE
        pltpu.sync_copy(acc.at[pl.ds(r0, ROWS_PER_TILE)],
                        out_hbm.at[c, pl.ds(r0, ROWS_PER_TILE)])

    return k(P, edges[0], edges[1], zeros_f)


BM = 1024
GRID = (N + BM - 1) // BM


def _tc_stage1(features, dego, W1):
    def body(f_ref, d_ref, w_ref, o_ref):
        deg = d_ref[0] + d_ref[1]
        ns = _norm(deg)
        o_ref[...] = jnp.dot(f_ref[...] * ns, w_ref[...],
                             preferred_element_type=jnp.float32)

    return pl.pallas_call(
        body,
        grid=(GRID,),
        in_specs=[
            pl.BlockSpec((BM, 128), lambda i: (i, 0)),
            pl.BlockSpec((2, BM, 1), lambda i: (0, i, 0)),
            pl.BlockSpec((128, 128), lambda i: (0, 0)),
        ],
        out_specs=pl.BlockSpec((BM, 128), lambda i: (i, 0)),
        out_shape=jax.ShapeDtypeStruct((N, 128), jnp.float32),
    )(features, dego, W1)


def _tc_stage2(a1p, dego, degi, W2, b1r):
    def body(a_ref, do_ref, di_ref, w_ref, b_ref, o_ref):
        a = a_ref[0] + a_ref[1]
        nd = _norm(di_ref[0] + di_ref[1])
        ns = _norm(do_ref[0] + do_ref[1])
        h = jnp.maximum(a * nd + b_ref[...], 0.0)
        o_ref[...] = jnp.dot(h * ns, w_ref[...],
                             preferred_element_type=jnp.float32)

    return pl.pallas_call(
        body,
        grid=(GRID,),
        in_specs=[
            pl.BlockSpec((2, BM, 128), lambda i: (0, i, 0)),
            pl.BlockSpec((2, BM, 1), lambda i: (0, i, 0)),
            pl.BlockSpec((2, BM, 1), lambda i: (0, i, 0)),
            pl.BlockSpec((128, 128), lambda i: (0, 0)),
            pl.BlockSpec((1, 128), lambda i: (0, 0)),
        ],
        out_specs=pl.BlockSpec((BM, 128), lambda i: (i, 0)),
        out_shape=jax.ShapeDtypeStruct((N, 128), jnp.float32),
    )(a1p, dego, degi, W2, b1r)


def _tc_stage3(a2p, degi, b2r):
    def body(a_ref, di_ref, b_ref, o_ref):
        a = a_ref[0, :, :64] + a_ref[1, :, :64]
        nd = _norm(di_ref[0] + di_ref[1])
        o_ref[...] = a * nd + b_ref[...]

    return pl.pallas_call(
        body,
        grid=(GRID,),
        in_specs=[
            pl.BlockSpec((2, BM, 128), lambda i: (0, i, 0)),
            pl.BlockSpec((2, BM, 1), lambda i: (0, i, 0)),
            pl.BlockSpec((1, 64), lambda i: (0, 0)),
        ],
        out_specs=pl.BlockSpec((BM, 64), lambda i: (i, 0)),
        out_shape=jax.ShapeDtypeStruct((N, 64), jnp.float32),
    )(a2p, degi, b2r)


def kernel(features, edge_index, W1, b1, W2, b2):
    src = edge_index[0]
    dst = edge_index[1]
    n_extra = E_PAD - E
    pad_scr = N + (jnp.arange(n_extra, dtype=jnp.int32) % (N_PAD_DEG - N))
    src_d = jnp.concatenate([src, pad_scr]).reshape(NW, CH, K)
    dst_p = jnp.concatenate([dst, pad_scr]).reshape(NW, CH, K)

    n2 = E_PAD2 - E
    pad_scr2 = N + (jnp.arange(n2, dtype=jnp.int32) % (N_PAD - N))
    src2 = jnp.concatenate(
        [src, jnp.zeros((n2,), jnp.int32)]).reshape(G_CHUNKS, 1, K2)
    dst2 = jnp.concatenate([dst, pad_scr2]).reshape(G_CHUNKS, 1, K2)
    edges2 = (src2, dst2)

    ones_col = jnp.ones((K,), jnp.float32)
    zeros_col = jnp.zeros((RPT_DEG,), jnp.float32)
    zeros128 = jnp.zeros((ROWS_PER_TILE, 128), jnp.float32)
    zeros64 = jnp.zeros((ROWS_PER_TILE, 64), jnp.float32)

    dego, degi = _sc_degrees(src_d, dst_p, ones_col, zeros_col)
    dego = dego[:, :, None]
    degi = degi[:, :, None]

    w2p = jnp.concatenate([W2, jnp.zeros((128, 64), jnp.float32)], axis=1)

    p1 = _tc_stage1(features, dego, W1)
    a1p = _sc_aggregate(p1, edges2, zeros128, 128)
    p2 = _tc_stage2(a1p, dego, degi, w2p, jnp.reshape(b1, (1, 128)))
    a2p = _sc_aggregate(p2, edges2, zeros128, 128)
    out = _tc_stage3(a2p, degi, jnp.reshape(b2, (1, 64)))
    return out
